# baseline (speedup 1.0000x reference)
import functools

import jax
import jax.numpy as jnp
from jax import lax
from jax.experimental import pallas as pl
from jax.experimental.pallas import tpu as pltpu
from jax.experimental.pallas import tpu_sc as plsc

_NC = 2
_NS = 16
_NW = _NC * _NS
_CL = 16
_LANES = 16
_NBUF = 4
_LA = 2


def kernel(x, pe):
    B, L, E = x.shape
    l_per_w = L // _NW
    n_chunks = l_per_w // _CL
    n_units = n_chunks * B
    unit = _CL * E
    vecs = unit // _LANES
    xf = x.reshape(B * L * E)
    pef = pe.reshape(pe.shape[0] * E)

    mesh = plsc.VectorSubcoreMesh(core_axis_name="c", subcore_axis_name="s")

    @functools.partial(
        pl.kernel,
        mesh=mesh,
        out_type=jax.ShapeDtypeStruct((B * L * E,), jnp.float32),
        scratch_types=[
            pltpu.VMEM((_NBUF, unit), jnp.float32),
            pltpu.VMEM((2, unit), jnp.float32),
        ]
        + [pltpu.SemaphoreType.DMA] * (2 * _NBUF + 2),
    )
    def sc_k(x_hbm, pe_hbm, o_hbm, xbuf, pebuf, *sems):
        lsem = sems[:_NBUF]
        ssem = sems[_NBUF : 2 * _NBUF]
        psem = sems[2 * _NBUF :]
        wid = lax.axis_index("s") * _NC + lax.axis_index("c")
        l0w = wid * l_per_w

        def x_off(u):
            c, b = divmod(u, B)
            return (b * L + l0w + c * _CL) * E

        def pe_off(c):
            return (l0w + c * _CL + 1) * E

        xcp = [None] * _NBUF
        scp = [None] * _NBUF
        pcp = [None] * 2

        pcp[0] = pltpu.async_copy(
            pe_hbm.at[pl.ds(pe_off(0), unit)], pebuf.at[0], psem[0]
        )
        for u in range(min(_LA, n_units)):
            s = u % _NBUF
            xcp[s] = pltpu.async_copy(
                x_hbm.at[pl.ds(x_off(u), unit)], xbuf.at[s], lsem[s]
            )

        for u in range(n_units):
            s = u % _NBUF
            c, b = divmod(u, B)

            v = u + _LA
            if v < n_units:
                sv = v % _NBUF
                if scp[sv] is not None:
                    scp[sv].wait()
                    scp[sv] = None
                xcp[sv] = pltpu.async_copy(
                    x_hbm.at[pl.ds(x_off(v), unit)], xbuf.at[sv], lsem[sv]
                )
            if b == 0 and c + 1 < n_chunks:
                pcp[(c + 1) % 2] = pltpu.async_copy(
                    pe_hbm.at[pl.ds(pe_off(c + 1), unit)],
                    pebuf.at[(c + 1) % 2],
                    psem[(c + 1) % 2],
                )

            xcp[s].wait()
            xcp[s] = None
            if b == 0:
                pcp[c % 2].wait()
                pcp[c % 2] = None

            pc = c % 2

            @plsc.parallel_loop(0, vecs, 1, unroll=8)
            def body(j):
                off = j * _LANES
                xbuf[s, pl.ds(off, _LANES)] = (
                    xbuf[s, pl.ds(off, _LANES)]
                    + pebuf[pc, pl.ds(off, _LANES)]
                )

            scp[s] = pltpu.async_copy(
                xbuf.at[s], o_hbm.at[pl.ds(x_off(u), unit)], ssem[s]
            )

        for s in range(_NBUF):
            if scp[s] is not None:
                scp[s].wait()

    return sc_k(xf, pef).reshape(B, L, E)


# speedup vs baseline: 1.0509x; 1.0509x over previous
import functools

import jax
import jax.numpy as jnp
from jax import lax
from jax.experimental import pallas as pl
from jax.experimental.pallas import tpu as pltpu
from jax.experimental.pallas import tpu_sc as plsc

_NC = 2
_NS = 16
_NW = _NC * _NS
_CL = 8
_LANES = 16


def kernel(x, pe):
    B, L, E = x.shape
    l_per_w = L // _NW
    n_chunks = l_per_w // _CL
    unit = _CL * E
    vecs = unit // _LANES
    nbuf = 2 * B
    xf = x.reshape(B * L * E)
    pef = pe.reshape(pe.shape[0] * E)

    mesh = plsc.VectorSubcoreMesh(core_axis_name="c", subcore_axis_name="s")

    @functools.partial(
        pl.kernel,
        mesh=mesh,
        out_type=jax.ShapeDtypeStruct((B * L * E,), jnp.float32),
        scratch_types=[
            pltpu.VMEM((nbuf, unit), jnp.float32),
            pltpu.VMEM((2, unit), jnp.float32),
        ]
        + [pltpu.SemaphoreType.DMA] * (2 * nbuf + 2),
    )
    def sc_k(x_hbm, pe_hbm, o_hbm, xbuf, pebuf, *sems):
        lsem = sems[:nbuf]
        ssem = sems[nbuf : 2 * nbuf]
        psem = sems[2 * nbuf :]
        wid = lax.axis_index("s") * _NC + lax.axis_index("c")
        l0w = wid * l_per_w

        def x_off(c, b):
            return (b * L + l0w + c * _CL) * E

        def pe_off(c):
            return (l0w + c * _CL + 1) * E

        xcp = [None] * nbuf
        scp = [None] * nbuf
        pcp = [None] * 2

        pcp[0] = pltpu.async_copy(
            pe_hbm.at[pl.ds(pe_off(0), unit)], pebuf.at[0], psem[0]
        )
        for b in range(B):
            xcp[b] = pltpu.async_copy(
                x_hbm.at[pl.ds(x_off(0, b), unit)], xbuf.at[b], lsem[b]
            )

        for c in range(n_chunks):
            g = (c % 2) * B
            if c + 1 < n_chunks:
                g2 = ((c + 1) % 2) * B
                p2 = (c + 1) % 2
                for b in range(B):
                    s2 = g2 + b
                    if scp[s2] is not None:
                        scp[s2].wait()
                        scp[s2] = None
                    xcp[s2] = pltpu.async_copy(
                        x_hbm.at[pl.ds(x_off(c + 1, b), unit)],
                        xbuf.at[s2],
                        lsem[s2],
                    )
                pcp[p2] = pltpu.async_copy(
                    pe_hbm.at[pl.ds(pe_off(c + 1), unit)],
                    pebuf.at[p2],
                    psem[p2],
                )

            for b in range(B):
                xcp[g + b].wait()
                xcp[g + b] = None
            pc = c % 2
            pcp[pc].wait()
            pcp[pc] = None

            @plsc.parallel_loop(0, vecs, 1, unroll=4)
            def body(j):
                off = j * _LANES
                pv = pebuf[pc, pl.ds(off, _LANES)]
                for b in range(B):
                    plsc.addupdate(xbuf.at[g + b, pl.ds(off, _LANES)], pv)

            for b in range(B):
                s = g + b
                scp[s] = pltpu.async_copy(
                    xbuf.at[s], o_hbm.at[pl.ds(x_off(c, b), unit)], ssem[s]
                )

        for s in range(nbuf):
            if scp[s] is not None:
                scp[s].wait()

    return sc_k(xf, pef).reshape(B, L, E)


# speedup vs baseline: 3.8672x; 3.6800x over previous
import jax
import jax.numpy as jnp
from jax.experimental import pallas as pl
from jax.experimental.pallas import tpu as pltpu

_BLK = 256


def _pe_add_kernel(x_ref, pe_ref, o_ref):
    o_ref[...] = x_ref[...] + pe_ref[...][None, :, :]


def kernel(x, pe):
    B, L, E = x.shape
    pe_rows = jax.lax.slice(pe, (1, 0), (1 + L, E))
    return pl.pallas_call(
        _pe_add_kernel,
        grid=(L // _BLK,),
        in_specs=[
            pl.BlockSpec((B, _BLK, E), lambda j: (0, j, 0)),
            pl.BlockSpec((_BLK, E), lambda j: (j, 0)),
        ],
        out_specs=pl.BlockSpec((B, _BLK, E), lambda j: (0, j, 0)),
        out_shape=jax.ShapeDtypeStruct((B, L, E), x.dtype),
        compiler_params=pltpu.CompilerParams(
            dimension_semantics=("parallel",),
        ),
    )(x, pe_rows)
